# baseline (speedup 1.0000x reference)
import functools

import jax
import jax.experimental.layout
import jax.numpy as jnp
from jax import lax
from jax.experimental import pallas as pl
from jax.experimental.pallas import tpu as pltpu
from jax.experimental.pallas import tpu_sc as plsc

NCOLS = 26
NROWS = 16384
TOKEN_DIM = 16
B_TOTAL = NROWS * NCOLS
PATTERN = 208

_info = plsc.get_sparse_core_info()
NW = _info.num_cores * _info.num_subcores
B_PER_W = B_TOTAL // NW
CHUNK = 1664
NCHUNK = B_PER_W // CHUNK
VREGS_PER_CHUNK = CHUNK // 16


def _make_kernel():
    mesh = plsc.VectorSubcoreMesh(core_axis_name="c", subcore_axis_name="s")

    @functools.partial(
        pl.kernel,
        mesh=mesh,
        compiler_params=pltpu.CompilerParams(use_tc_tiling_on_sc=False),
        out_type=jax.ShapeDtypeStruct((NROWS, NCOLS, TOKEN_DIM), jnp.float32),
        scratch_types=[
            pltpu.VMEM((CHUNK,), jnp.int32),
            pltpu.VMEM((CHUNK,), jnp.int32),
            pltpu.VMEM((CHUNK,), jnp.int32),
            pltpu.VMEM((CHUNK,), jnp.int32),
            pltpu.VMEM((CHUNK, TOKEN_DIM), jnp.float32),
            pltpu.VMEM((CHUNK, TOKEN_DIM), jnp.float32),
            pltpu.VMEM((PATTERN,), jnp.int32),
            pltpu.SemaphoreType.DMA,
            pltpu.SemaphoreType.DMA,
        ],
    )
    def emb_kernel(x_hbm, pat_hbm, table_hbm, out_hbm,
                   xbuf0, xbuf1, idxbuf0, idxbuf1, rows0, rows1,
                   pat_v, sem0, sem1):
        wid = lax.axis_index("c") * _info.num_subcores + lax.axis_index("s")
        base = wid * B_PER_W

        xbufs = (xbuf0, xbuf1)
        idxbufs = (idxbuf0, idxbuf1)
        rowbufs = (rows0, rows1)
        sems = (sem0, sem1)

        pltpu.sync_copy(pat_hbm, pat_v)

        def start_chunk(c, nb):
            gb = base + c * CHUNK
            xb, ib = xbufs[nb], idxbufs[nb]
            pltpu.sync_copy(x_hbm.at[pl.ds(gb, CHUNK)], xb)

            def add_body(v, carry):
                ph = 16 * lax.rem(v, 13)
                ib[pl.ds(16 * v, 16)] = xb[pl.ds(16 * v, 16)] + pat_v[pl.ds(ph, 16)]
                return carry

            lax.fori_loop(0, VREGS_PER_CHUNK, add_body, 0)
            return pltpu.async_copy(table_hbm.at[ib], rowbufs[nb], sems[nb])

        handle = start_chunk(0, 0)
        for c in range(NCHUNK):
            nb = c % 2
            nxt = None
            if c + 1 < NCHUNK:
                nxt = start_chunk(c + 1, 1 - nb)
            handle.wait()
            row0 = (base + c * CHUNK) // NCOLS

            def out_row(r, carry):
                pltpu.sync_copy(rowbufs[nb].at[pl.ds(NCOLS * r, NCOLS)],
                                out_hbm.at[row0 + r])
                return carry

            lax.fori_loop(0, CHUNK // NCOLS, out_row, 0)
            handle = nxt

    return emb_kernel


_emb = _make_kernel()


@jax.jit
def kernel(x_cat, category_offsets, table):
    x_flat = x_cat.reshape(B_TOTAL).astype(jnp.int32)
    pat = jnp.tile(category_offsets.astype(jnp.int32), PATTERN // NCOLS)
    out = _emb(x_flat, pat, table)
    lay = jax.experimental.layout.Layout(major_to_minor=(0, 1, 2),
                                         tiling=((16,),))
    return jax.experimental.layout.with_layout_constraint(out, lay)
